# baseline (speedup 1.0000x reference)
import jax
import jax.numpy as jnp
from jax.experimental import pallas as pl
from jax.experimental.pallas import tpu as pltpu

N = 100000


def _body(s_ref, b_ref, x_ref, o_ref):
    o_ref[...] = x_ref[...] * s_ref[0] + b_ref[0]


@jax.jit
def _scale_shift(x, s1, b1):
    B = 25600
    return pl.pallas_call(
        _body,
        out_shape=jax.ShapeDtypeStruct((N,), jnp.float32),
        grid=((N + B - 1) // B,),
        in_specs=[
            pl.BlockSpec(memory_space=pltpu.SMEM),
            pl.BlockSpec(memory_space=pltpu.SMEM),
            pl.BlockSpec((B,), lambda i: (i,)),
        ],
        out_specs=pl.BlockSpec((B,), lambda i: (i,)),
    )(s1, b1, x)


def kernel(x, head, scale, shift):
    s1 = jnp.reshape(scale, (1,))
    b1 = jnp.reshape(shift, (1,))
    return _scale_shift(x, s1, b1)


# speedup vs baseline: 1.4509x; 1.4509x over previous
import jax
import jax.numpy as jnp
from jax.experimental import pallas as pl
from jax.experimental.pallas import tpu as pltpu

N = 100000


def _body(s_ref, b_ref, x_ref, o_ref):
    o_ref[...] = x_ref[...] * s_ref[0] + b_ref[0]


@jax.jit
def _scale_shift(x, s1, b1):
    B = 50176
    return pl.pallas_call(
        _body,
        out_shape=jax.ShapeDtypeStruct((N,), jnp.float32),
        grid=((N + B - 1) // B,),
        in_specs=[
            pl.BlockSpec(memory_space=pltpu.SMEM),
            pl.BlockSpec(memory_space=pltpu.SMEM),
            pl.BlockSpec((B,), lambda i: (i,)),
        ],
        out_specs=pl.BlockSpec((B,), lambda i: (i,)),
    )(s1, b1, x)


def kernel(x, head, scale, shift):
    s1 = jnp.reshape(scale, (1,))
    b1 = jnp.reshape(shift, (1,))
    return _scale_shift(x, s1, b1)


# speedup vs baseline: 1.4610x; 1.0069x over previous
import jax
import jax.numpy as jnp
from jax.experimental import pallas as pl
from jax.experimental.pallas import tpu as pltpu

N = 100000
BLOCK = 50176


def _body(s_ref, b_ref, x_ref, o_ref):
    o_ref[...] = x_ref[...] * s_ref[0] + b_ref[0]


@jax.jit
def _scale_shift(x, s1, b1):
    return pl.pallas_call(
        _body,
        out_shape=jax.ShapeDtypeStruct((N,), jnp.float32),
        grid=((N + BLOCK - 1) // BLOCK,),
        in_specs=[
            pl.BlockSpec(memory_space=pltpu.SMEM),
            pl.BlockSpec(memory_space=pltpu.SMEM),
            pl.BlockSpec((BLOCK,), lambda i: (i,)),
        ],
        out_specs=pl.BlockSpec((BLOCK,), lambda i: (i,)),
    )(s1, b1, x)


def kernel(x, head, scale, shift):
    s1 = jnp.reshape(scale, (1,))
    b1 = jnp.reshape(shift, (1,))
    return _scale_shift(x, s1, b1)
